# Initial kernel scaffold; baseline (speedup 1.0000x reference)
import functools

import jax
import jax.numpy as jnp
from jax import lax
from jax.experimental import pallas as pl
from jax.experimental.pallas import tpu as pltpu
from jax.experimental.pallas import tpu_sc as plsc

N = 10000
E = 320000
D = 128
G = 64
C = 10

NC = 2
NS = 16
NW = NC * NS
LW = 16

CH = 128
PER_W = 10240
E_PAD = PER_W * NW
N_PAD = 10240
ROWS_PER_SUB = N_PAD // NS

_vmesh = plsc.VectorSubcoreMesh(core_axis_name="c", subcore_axis_name="s")


@functools.partial(
    pl.kernel,
    out_type=jax.ShapeDtypeStruct((NC, N_PAD, D), jnp.float32),
    mesh=_vmesh,
    scratch_types=[
        pltpu.VMEM((CH,), jnp.int32),
        pltpu.VMEM((CH,), jnp.int32),
        pltpu.VMEM((CH, D), jnp.float32),
        pltpu.VMEM_SHARED((N_PAD, D), jnp.float32),
        pltpu.SemaphoreType.DMA,
    ],
)
def _sc_agg(h_hbm, src_hbm, dst_hbm, zeros_hbm, out_hbm, src_v, dst_v, rows_v, acc, sem):
    c = lax.axis_index("c")
    s = lax.axis_index("s")
    r0 = s * ROWS_PER_SUB
    pltpu.sync_copy(zeros_hbm.at[pl.ds(r0, ROWS_PER_SUB)], acc.at[pl.ds(r0, ROWS_PER_SUB)])
    plsc.subcore_barrier()
    base = (c * NS + s) * PER_W

    @pl.loop(0, PER_W, step=CH)
    def _(i):
        e0 = base + i
        pltpu.sync_copy(src_hbm.at[pl.ds(e0, CH)], src_v)
        pltpu.async_copy(h_hbm.at[src_v], rows_v, sem).wait()
        pltpu.sync_copy(dst_hbm.at[pl.ds(e0, CH)], dst_v)
        pltpu.sync_copy(rows_v, acc.at[dst_v], add=True)

    plsc.subcore_barrier()
    pltpu.sync_copy(acc.at[pl.ds(r0, ROWS_PER_SUB)], out_hbm.at[c, pl.ds(r0, ROWS_PER_SUB)])


@functools.partial(
    pl.kernel,
    out_type=jax.ShapeDtypeStruct((NC, N_PAD, LW), jnp.float32),
    mesh=_vmesh,
    scratch_types=[
        pltpu.VMEM((CH,), jnp.int32),
        pltpu.VMEM((CH, LW), jnp.float32),
        pltpu.VMEM_SHARED((N_PAD, LW), jnp.float32),
        pltpu.SemaphoreType.DMA,
    ],
)
def _sc_cnt(dst_hbm, ones_hbm, zeros16_hbm, out_hbm, dst_v, ones_v, acc, sem):
    c = lax.axis_index("c")
    s = lax.axis_index("s")
    r0 = s * ROWS_PER_SUB
    pltpu.sync_copy(zeros16_hbm.at[pl.ds(r0, ROWS_PER_SUB)], acc.at[pl.ds(r0, ROWS_PER_SUB)])
    pltpu.sync_copy(ones_hbm, ones_v)
    plsc.subcore_barrier()
    base = (c * NS + s) * PER_W

    @pl.loop(0, PER_W, step=CH)
    def _(i):
        pltpu.sync_copy(dst_hbm.at[pl.ds(base + i, CH)], dst_v)
        pltpu.sync_copy(ones_v, acc.at[dst_v], add=True)

    plsc.subcore_barrier()
    pltpu.sync_copy(acc.at[pl.ds(r0, ROWS_PER_SUB)], out_hbm.at[c, pl.ds(r0, ROWS_PER_SUB)])


_NFULL = N // CH
_TAIL = N - _NFULL * CH
_GPS = G // NS


@functools.partial(
    pl.kernel,
    out_type=jax.ShapeDtypeStruct((NC, G, D), jnp.float32),
    mesh=_vmesh,
    scratch_types=[
        pltpu.VMEM((CH,), jnp.int32),
        pltpu.VMEM((CH, D), jnp.float32),
        pltpu.VMEM((_TAIL,), jnp.int32),
        pltpu.VMEM((_TAIL, D), jnp.float32),
        pltpu.VMEM_SHARED((G, D), jnp.float32),
        pltpu.SemaphoreType.DMA,
    ],
)
def _sc_pool(h_hbm, batch_hbm, zeros_hbm, out_hbm, b_v, rows_v, bt_v, rowst_v, acc, sem):
    c = lax.axis_index("c")
    s = lax.axis_index("s")
    wid = c * NS + s
    g0 = s * _GPS
    pltpu.sync_copy(zeros_hbm.at[pl.ds(0, _GPS)], acc.at[pl.ds(g0, _GPS)])
    plsc.subcore_barrier()

    @pl.loop(wid, _NFULL, step=NW)
    def _(j):
        n0 = j * CH
        pltpu.sync_copy(batch_hbm.at[pl.ds(n0, CH)], b_v)
        pltpu.sync_copy(h_hbm.at[pl.ds(n0, CH)], rows_v)
        pltpu.sync_copy(rows_v, acc.at[b_v], add=True)

    @pl.when(wid == NW - 1)
    def _():
        n0 = _NFULL * CH
        pltpu.sync_copy(batch_hbm.at[pl.ds(n0, _TAIL)], bt_v)
        pltpu.sync_copy(h_hbm.at[pl.ds(n0, _TAIL)], rowst_v)
        pltpu.sync_copy(rowst_v, acc.at[bt_v], add=True)

    plsc.subcore_barrier()
    pltpu.sync_copy(acc.at[pl.ds(g0, _GPS)], out_hbm.at[c, pl.ds(g0, _GPS)])


TN = 2000


def _tc_layer_body(h_ref, agg_ref, cnt_ref, wl_ref, b_ref, wr_ref, o_ref, *, relu):
    agg = agg_ref[0] + agg_ref[1]
    cnt = cnt_ref[0, :, 0:1] + cnt_ref[1, :, 0:1]
    inv = 1.0 / jnp.maximum(cnt, 1.0)
    dn = (((1,), (1,)), ((), ()))
    y = lax.dot_general(agg * inv, wl_ref[...], dn, preferred_element_type=jnp.float32)
    y = y + b_ref[...]
    y = y + lax.dot_general(h_ref[...], wr_ref[...], dn, preferred_element_type=jnp.float32)
    o_ref[...] = jnp.maximum(y, 0.0) if relu else y


def _tc_layer(h, agg2, cnt2, Wl, b, Wr, relu):
    return pl.pallas_call(
        functools.partial(_tc_layer_body, relu=relu),
        grid=(N // TN,),
        in_specs=[
            pl.BlockSpec((TN, D), lambda i: (i, 0)),
            pl.BlockSpec((NC, TN, D), lambda i: (0, i, 0)),
            pl.BlockSpec((NC, TN, LW), lambda i: (0, i, 0)),
            pl.BlockSpec((D, D), lambda i: (0, 0)),
            pl.BlockSpec((1, D), lambda i: (0, 0)),
            pl.BlockSpec((D, D), lambda i: (0, 0)),
        ],
        out_specs=pl.BlockSpec((TN, D), lambda i: (i, 0)),
        out_shape=jax.ShapeDtypeStruct((N, D), jnp.float32),
    )(h, agg2, cnt2, Wl, b.reshape(1, D), Wr)


def _tc_final_body(p_ref, w_ref, b_ref, o_ref):
    p = p_ref[0] + p_ref[1]
    dn = (((1,), (1,)), ((), ()))
    y = lax.dot_general(p, w_ref[...], dn, preferred_element_type=jnp.float32)
    o_ref[...] = y + b_ref[...]


def _tc_final(pooled2, lin_W, lin_b):
    return pl.pallas_call(
        _tc_final_body,
        out_shape=jax.ShapeDtypeStruct((G, C), jnp.float32),
    )(pooled2, lin_W, lin_b.reshape(1, C))


def kernel(x, edge_index, batch, W1l, b1, W1r, W2l, b2, W2r, W3l, b3, W3r,
           W4l, b4, W4r, W5l, b5, W5r, W6l, b6, W6r, W7l, b7, W7r, lin_W, lin_b):
    src = edge_index[0]
    dst = edge_index[1]
    pad = E_PAD - E
    src_p = jnp.concatenate([src, jnp.zeros((pad,), jnp.int32)])
    dst_p = jnp.concatenate([dst, jnp.full((pad,), N, jnp.int32)])
    zeros = jnp.zeros((N_PAD, D), jnp.float32)
    zeros16 = jnp.zeros((N_PAD, LW), jnp.float32)
    ones16 = jnp.ones((CH, LW), jnp.float32)

    cnt2 = _sc_cnt(dst_p, ones16, zeros16)

    convs = [(W1l, b1, W1r), (W2l, b2, W2r), (W3l, b3, W3r), (W4l, b4, W4r),
             (W5l, b5, W5r), (W6l, b6, W6r), (W7l, b7, W7r)]
    h = x
    for i, (Wl, b, Wr) in enumerate(convs):
        agg2 = _sc_agg(h, src_p, dst_p, zeros)
        h = _tc_layer(h, agg2, cnt2, Wl, b, Wr, relu=(i < 6))

    pooled2 = _sc_pool(h, batch, zeros)
    return _tc_final(pooled2, lin_W, lin_b)


# speedup vs baseline: 2.7145x; 2.7145x over previous
import functools

import jax
import jax.numpy as jnp
from jax import lax
from jax.experimental import pallas as pl
from jax.experimental.pallas import tpu as pltpu
from jax.experimental.pallas import tpu_sc as plsc

N = 10000
E = 320000
D = 128
G = 64
C = 10

NC = 2
NS = 16
NW = NC * NS
LW = 16

CH = 128
PER_W = 10240
E_PAD = PER_W * NW
N_PAD = 10240
ROWS_PER_SUB = N_PAD // NS

_vmesh = plsc.VectorSubcoreMesh(core_axis_name="c", subcore_axis_name="s")


@functools.partial(
    pl.kernel,
    out_type=jax.ShapeDtypeStruct((NC, N_PAD, D), jnp.float32),
    mesh=_vmesh,
    scratch_types=[
        pltpu.VMEM((CH,), jnp.int32),
        pltpu.VMEM((CH,), jnp.int32),
        pltpu.VMEM((CH, D), jnp.float32),
        pltpu.VMEM_SHARED((N_PAD, D), jnp.float32),
        pltpu.SemaphoreType.DMA,
    ],
)
def _sc_agg(h_hbm, src_hbm, dst_hbm, zeros_hbm, out_hbm, src_v, dst_v, rows_v, acc, sem):
    c = lax.axis_index("c")
    s = lax.axis_index("s")
    r0 = s * ROWS_PER_SUB
    pltpu.sync_copy(zeros_hbm.at[pl.ds(r0, ROWS_PER_SUB)], acc.at[pl.ds(r0, ROWS_PER_SUB)])
    plsc.subcore_barrier()
    base = (c * NS + s) * PER_W

    @pl.loop(0, PER_W, step=CH)
    def _(i):
        e0 = base + i
        pltpu.sync_copy(src_hbm.at[pl.ds(e0, CH)], src_v)
        pltpu.async_copy(h_hbm.at[src_v], rows_v, sem).wait()
        pltpu.sync_copy(dst_hbm.at[pl.ds(e0, CH)], dst_v)
        pltpu.sync_copy(rows_v, acc.at[dst_v], add=True)

    plsc.subcore_barrier()
    pltpu.sync_copy(acc.at[pl.ds(r0, ROWS_PER_SUB)], out_hbm.at[c, pl.ds(r0, ROWS_PER_SUB)])


@functools.partial(
    pl.kernel,
    out_type=jax.ShapeDtypeStruct((NC, N_PAD, D), jnp.float32),
    mesh=_vmesh,
    scratch_types=[
        pltpu.VMEM((CH,), jnp.int32),
        pltpu.VMEM((CH, D), jnp.float32),
        pltpu.VMEM_SHARED((N_PAD, D), jnp.float32),
        pltpu.SemaphoreType.DMA,
    ],
)
def _sc_cnt(dst_hbm, ones_hbm, zeros_hbm, out_hbm, dst_v, ones_v, acc, sem):
    c = lax.axis_index("c")
    s = lax.axis_index("s")
    r0 = s * ROWS_PER_SUB
    pltpu.sync_copy(zeros_hbm.at[pl.ds(r0, ROWS_PER_SUB)], acc.at[pl.ds(r0, ROWS_PER_SUB)])
    pltpu.sync_copy(ones_hbm, ones_v)
    plsc.subcore_barrier()
    base = (c * NS + s) * PER_W

    @pl.loop(0, PER_W, step=CH)
    def _(i):
        pltpu.sync_copy(dst_hbm.at[pl.ds(base + i, CH)], dst_v)
        pltpu.sync_copy(ones_v, acc.at[dst_v], add=True)

    plsc.subcore_barrier()
    pltpu.sync_copy(acc.at[pl.ds(r0, ROWS_PER_SUB)], out_hbm.at[c, pl.ds(r0, ROWS_PER_SUB)])


def _tc_inv_body(cnt_ref, o_ref):
    cnt = cnt_ref[0, :, 0:1] + cnt_ref[1, :, 0:1]
    o_ref[...] = 1.0 / jnp.maximum(cnt, 1.0)


def _tc_inv(cnt2):
    return pl.pallas_call(
        _tc_inv_body,
        grid=(N // TN,),
        in_specs=[pl.BlockSpec((NC, TN, D), lambda i: (0, i, 0))],
        out_specs=pl.BlockSpec((TN, 1), lambda i: (i, 0)),
        out_shape=jax.ShapeDtypeStruct((N, 1), jnp.float32),
    )(cnt2)


_NFULL = N // CH
_TAIL = N - _NFULL * CH
_GPS = G // NS


@functools.partial(
    pl.kernel,
    out_type=jax.ShapeDtypeStruct((NC, G, D), jnp.float32),
    mesh=_vmesh,
    scratch_types=[
        pltpu.VMEM((CH,), jnp.int32),
        pltpu.VMEM((CH, D), jnp.float32),
        pltpu.VMEM((_TAIL,), jnp.int32),
        pltpu.VMEM((_TAIL, D), jnp.float32),
        pltpu.VMEM_SHARED((G, D), jnp.float32),
        pltpu.SemaphoreType.DMA,
    ],
)
def _sc_pool(h_hbm, batch_hbm, zeros_hbm, out_hbm, b_v, rows_v, bt_v, rowst_v, acc, sem):
    c = lax.axis_index("c")
    s = lax.axis_index("s")
    wid = c * NS + s
    g0 = s * _GPS
    pltpu.sync_copy(zeros_hbm.at[pl.ds(0, _GPS)], acc.at[pl.ds(g0, _GPS)])
    plsc.subcore_barrier()

    @pl.loop(wid, _NFULL, step=NW)
    def _(j):
        n0 = j * CH
        pltpu.sync_copy(batch_hbm.at[pl.ds(n0, CH)], b_v)
        pltpu.sync_copy(h_hbm.at[pl.ds(n0, CH)], rows_v)
        pltpu.sync_copy(rows_v, acc.at[b_v], add=True)

    @pl.when(wid == NW - 1)
    def _():
        n0 = _NFULL * CH
        pltpu.sync_copy(batch_hbm.at[pl.ds(n0, _TAIL)], bt_v)
        pltpu.sync_copy(h_hbm.at[pl.ds(n0, _TAIL)], rowst_v)
        pltpu.sync_copy(rowst_v, acc.at[bt_v], add=True)

    plsc.subcore_barrier()
    pltpu.sync_copy(acc.at[pl.ds(g0, _GPS)], out_hbm.at[c, pl.ds(g0, _GPS)])


TN = 2000


def _tc_layer_body(h_ref, agg_ref, inv_ref, wl_ref, b_ref, wr_ref, o_ref, *, relu):
    agg = agg_ref[0] + agg_ref[1]
    inv = inv_ref[...]
    dn = (((1,), (1,)), ((), ()))
    y = lax.dot_general(agg * inv, wl_ref[...], dn, preferred_element_type=jnp.float32)
    y = y + b_ref[...]
    y = y + lax.dot_general(h_ref[...], wr_ref[...], dn, preferred_element_type=jnp.float32)
    o_ref[...] = jnp.maximum(y, 0.0) if relu else y


def _tc_layer(h, agg2, inv, Wl, b, Wr, relu):
    return pl.pallas_call(
        functools.partial(_tc_layer_body, relu=relu),
        grid=(N // TN,),
        in_specs=[
            pl.BlockSpec((TN, D), lambda i: (i, 0)),
            pl.BlockSpec((NC, TN, D), lambda i: (0, i, 0)),
            pl.BlockSpec((TN, 1), lambda i: (i, 0)),
            pl.BlockSpec((D, D), lambda i: (0, 0)),
            pl.BlockSpec((1, D), lambda i: (0, 0)),
            pl.BlockSpec((D, D), lambda i: (0, 0)),
        ],
        out_specs=pl.BlockSpec((TN, D), lambda i: (i, 0)),
        out_shape=jax.ShapeDtypeStruct((N, D), jnp.float32),
    )(h, agg2, inv, Wl, b.reshape(1, D), Wr)


def _tc_final_body(p_ref, w_ref, b_ref, o_ref):
    p = p_ref[0] + p_ref[1]
    dn = (((1,), (1,)), ((), ()))
    y = lax.dot_general(p, w_ref[...], dn, preferred_element_type=jnp.float32)
    o_ref[...] = y + b_ref[...]


def _tc_final(pooled2, lin_W, lin_b):
    return pl.pallas_call(
        _tc_final_body,
        out_shape=jax.ShapeDtypeStruct((G, C), jnp.float32),
    )(pooled2, lin_W, lin_b.reshape(1, C))


def kernel(x, edge_index, batch, W1l, b1, W1r, W2l, b2, W2r, W3l, b3, W3r,
           W4l, b4, W4r, W5l, b5, W5r, W6l, b6, W6r, W7l, b7, W7r, lin_W, lin_b):
    src = edge_index[0]
    dst = edge_index[1]
    pad = E_PAD - E
    src_p = jnp.concatenate([src, jnp.zeros((pad,), jnp.int32)])
    dst_p = jnp.concatenate([dst, jnp.full((pad,), N, jnp.int32)])
    zeros = jnp.zeros((N_PAD, D), jnp.float32)
    ones = jnp.ones((CH, D), jnp.float32)

    cnt2 = _sc_cnt(dst_p, ones, zeros)
    inv = _tc_inv(cnt2)

    convs = [(W1l, b1, W1r), (W2l, b2, W2r), (W3l, b3, W3r), (W4l, b4, W4r),
             (W5l, b5, W5r), (W6l, b6, W6r), (W7l, b7, W7r)]
    h = x
    for i, (Wl, b, Wr) in enumerate(convs):
        agg2 = _sc_agg(h, src_p, dst_p, zeros)
        h = _tc_layer(h, agg2, inv, Wl, b, Wr, relu=(i < 6))

    pooled2 = _sc_pool(h, batch, zeros)
    return _tc_final(pooled2, lin_W, lin_b)


# speedup vs baseline: 3.0938x; 1.1397x over previous
import functools

import jax
import jax.numpy as jnp
from jax import lax
from jax.experimental import pallas as pl
from jax.experimental.pallas import tpu as pltpu
from jax.experimental.pallas import tpu_sc as plsc

N = 10000
E = 320000
D = 128
G = 64
C = 10

NC = 2
NS = 16
NW = NC * NS
LW = 16

CH = 128
PER_W = 10240
E_PAD = PER_W * NW
N_PAD = 10240
ROWS_PER_SUB = N_PAD // NS

_vmesh = plsc.VectorSubcoreMesh(core_axis_name="c", subcore_axis_name="s")


NCH = PER_W // CH
NCHT = E_PAD // CH


@functools.partial(
    pl.kernel,
    out_type=jax.ShapeDtypeStruct((NC, N_PAD, D), jnp.float32),
    mesh=_vmesh,
    scratch_types=[
        pltpu.VMEM((2, CH), jnp.int32),
        pltpu.VMEM((2, CH), jnp.int32),
        pltpu.VMEM((CH, D), jnp.float32),
        pltpu.VMEM((CH, D), jnp.float32),
        pltpu.VMEM_SHARED((N_PAD, D), jnp.float32),
        pltpu.SemaphoreType.DMA,
        pltpu.SemaphoreType.DMA,
        pltpu.SemaphoreType.DMA,
        pltpu.SemaphoreType.DMA,
    ],
)
def _sc_agg(h_hbm, ids_hbm, zeros_hbm, out_hbm, idx0, idx1, rows0, rows1, acc,
            isem0, isem1, gsem0, gsem1):
    c = lax.axis_index("c")
    s = lax.axis_index("s")
    r0 = s * ROWS_PER_SUB
    pltpu.sync_copy(zeros_hbm.at[pl.ds(r0, ROWS_PER_SUB)], acc.at[pl.ds(r0, ROWS_PER_SUB)])
    plsc.subcore_barrier()
    b0 = (c * NS + s) * NCH

    idx = (idx0, idx1)
    rows = (rows0, rows1)
    isem = (isem0, isem1)
    gsem = (gsem0, gsem1)

    pltpu.sync_copy(ids_hbm.at[b0], idx[0])
    pltpu.async_copy(ids_hbm.at[b0 + 1], idx[1], isem[1])
    pltpu.async_copy(h_hbm.at[idx[0].at[0]], rows[0], gsem[0])

    def step(k, p, has_next, has_next2):
        q = 1 - p
        pltpu.make_async_copy(h_hbm.at[idx[p].at[0]], rows[p], gsem[p]).wait()
        if has_next:
            pltpu.make_async_copy(ids_hbm.at[k + 1], idx[q], isem[q]).wait()
            pltpu.async_copy(h_hbm.at[idx[q].at[0]], rows[q], gsem[q])
        pltpu.sync_copy(rows[p], acc.at[idx[p].at[1]], add=True)
        if has_next2:
            pltpu.async_copy(ids_hbm.at[k + 2], idx[p], isem[p])

    @pl.loop(0, NCH - 4, step=2)
    def _(i):
        step(b0 + i, 0, True, True)
        step(b0 + i + 1, 1, True, True)

    step(b0 + NCH - 4, 0, True, True)
    step(b0 + NCH - 3, 1, True, True)
    step(b0 + NCH - 2, 0, True, False)
    step(b0 + NCH - 1, 1, False, False)

    plsc.subcore_barrier()
    pltpu.sync_copy(acc.at[pl.ds(r0, ROWS_PER_SUB)], out_hbm.at[c, pl.ds(r0, ROWS_PER_SUB)])


@functools.partial(
    pl.kernel,
    out_type=jax.ShapeDtypeStruct((NC, N_PAD, D), jnp.float32),
    mesh=_vmesh,
    scratch_types=[
        pltpu.VMEM((2, CH), jnp.int32),
        pltpu.VMEM((2, CH), jnp.int32),
        pltpu.VMEM((CH, D), jnp.float32),
        pltpu.VMEM_SHARED((N_PAD, D), jnp.float32),
        pltpu.SemaphoreType.DMA,
        pltpu.SemaphoreType.DMA,
    ],
)
def _sc_cnt(ids_hbm, ones_hbm, zeros_hbm, out_hbm, idx0, idx1, ones_v, acc,
            isem0, isem1):
    c = lax.axis_index("c")
    s = lax.axis_index("s")
    r0 = s * ROWS_PER_SUB
    pltpu.sync_copy(zeros_hbm.at[pl.ds(r0, ROWS_PER_SUB)], acc.at[pl.ds(r0, ROWS_PER_SUB)])
    pltpu.sync_copy(ones_hbm, ones_v)
    plsc.subcore_barrier()
    b0 = (c * NS + s) * NCH

    idx = (idx0, idx1)
    isem = (isem0, isem1)
    pltpu.sync_copy(ids_hbm.at[b0], idx[0])
    pltpu.async_copy(ids_hbm.at[b0 + 1], idx[1], isem[1])

    def step(k, p, has_next, has_next2):
        q = 1 - p
        if has_next:
            pltpu.make_async_copy(ids_hbm.at[k + 1], idx[q], isem[q]).wait()
        pltpu.sync_copy(ones_v, acc.at[idx[p].at[1]], add=True)
        if has_next2:
            pltpu.async_copy(ids_hbm.at[k + 2], idx[p], isem[p])

    @pl.loop(0, NCH - 2, step=2)
    def _(i):
        step(b0 + i, 0, True, True)
        step(b0 + i + 1, 1, True, True)

    step(b0 + NCH - 2, 0, True, False)
    step(b0 + NCH - 1, 1, False, False)

    plsc.subcore_barrier()
    pltpu.sync_copy(acc.at[pl.ds(r0, ROWS_PER_SUB)], out_hbm.at[c, pl.ds(r0, ROWS_PER_SUB)])


def _tc_inv_body(cnt_ref, o_ref):
    cnt = cnt_ref[0, :, 0:1] + cnt_ref[1, :, 0:1]
    o_ref[...] = 1.0 / jnp.maximum(cnt, 1.0)


def _tc_inv(cnt2):
    return pl.pallas_call(
        _tc_inv_body,
        grid=(N // TN,),
        in_specs=[pl.BlockSpec((NC, TN, D), lambda i: (0, i, 0))],
        out_specs=pl.BlockSpec((TN, 1), lambda i: (i, 0)),
        out_shape=jax.ShapeDtypeStruct((N, 1), jnp.float32),
    )(cnt2)


_NFULL = N // CH
_TAIL = N - _NFULL * CH
_GPS = G // NS


@functools.partial(
    pl.kernel,
    out_type=jax.ShapeDtypeStruct((NC, G, D), jnp.float32),
    mesh=_vmesh,
    scratch_types=[
        pltpu.VMEM((CH,), jnp.int32),
        pltpu.VMEM((CH, D), jnp.float32),
        pltpu.VMEM((_TAIL,), jnp.int32),
        pltpu.VMEM((_TAIL, D), jnp.float32),
        pltpu.VMEM_SHARED((G, D), jnp.float32),
        pltpu.SemaphoreType.DMA,
    ],
)
def _sc_pool(h_hbm, batch_hbm, zeros_hbm, out_hbm, b_v, rows_v, bt_v, rowst_v, acc, sem):
    c = lax.axis_index("c")
    s = lax.axis_index("s")
    wid = c * NS + s
    g0 = s * _GPS
    pltpu.sync_copy(zeros_hbm.at[pl.ds(0, _GPS)], acc.at[pl.ds(g0, _GPS)])
    plsc.subcore_barrier()

    @pl.loop(wid, _NFULL, step=NW)
    def _(j):
        n0 = j * CH
        pltpu.sync_copy(batch_hbm.at[pl.ds(n0, CH)], b_v)
        pltpu.sync_copy(h_hbm.at[pl.ds(n0, CH)], rows_v)
        pltpu.sync_copy(rows_v, acc.at[b_v], add=True)

    @pl.when(wid == NW - 1)
    def _():
        n0 = _NFULL * CH
        pltpu.sync_copy(batch_hbm.at[pl.ds(n0, _TAIL)], bt_v)
        pltpu.sync_copy(h_hbm.at[pl.ds(n0, _TAIL)], rowst_v)
        pltpu.sync_copy(rowst_v, acc.at[bt_v], add=True)

    plsc.subcore_barrier()
    pltpu.sync_copy(acc.at[pl.ds(g0, _GPS)], out_hbm.at[c, pl.ds(g0, _GPS)])


TN = 2000


def _tc_right_body(h_ref, wr_ref, b_ref, o_ref):
    dn = (((1,), (1,)), ((), ()))
    y = lax.dot_general(h_ref[...], wr_ref[...], dn, preferred_element_type=jnp.float32)
    o_ref[...] = y + b_ref[...]


def _tc_right(h, Wr, b):
    return pl.pallas_call(
        _tc_right_body,
        grid=(N // TN,),
        in_specs=[
            pl.BlockSpec((TN, D), lambda i: (i, 0)),
            pl.BlockSpec((D, D), lambda i: (0, 0)),
            pl.BlockSpec((1, D), lambda i: (0, 0)),
        ],
        out_specs=pl.BlockSpec((TN, D), lambda i: (i, 0)),
        out_shape=jax.ShapeDtypeStruct((N, D), jnp.float32),
    )(h, Wr, b.reshape(1, D))


def _tc_combine_body(agg_ref, inv_ref, wl_ref, r_ref, o_ref, *, relu):
    agg = agg_ref[0] + agg_ref[1]
    inv = inv_ref[...]
    dn = (((1,), (1,)), ((), ()))
    y = lax.dot_general(agg * inv, wl_ref[...], dn, preferred_element_type=jnp.float32)
    y = y + r_ref[...]
    o_ref[...] = jnp.maximum(y, 0.0) if relu else y


def _tc_combine(agg2, inv, Wl, r, relu):
    return pl.pallas_call(
        functools.partial(_tc_combine_body, relu=relu),
        grid=(N // TN,),
        in_specs=[
            pl.BlockSpec((NC, TN, D), lambda i: (0, i, 0)),
            pl.BlockSpec((TN, 1), lambda i: (i, 0)),
            pl.BlockSpec((D, D), lambda i: (0, 0)),
            pl.BlockSpec((TN, D), lambda i: (i, 0)),
        ],
        out_specs=pl.BlockSpec((TN, D), lambda i: (i, 0)),
        out_shape=jax.ShapeDtypeStruct((N, D), jnp.float32),
    )(agg2, inv, Wl, r)


def _tc_final_body(p_ref, w_ref, b_ref, o_ref):
    p = p_ref[0] + p_ref[1]
    dn = (((1,), (1,)), ((), ()))
    y = lax.dot_general(p, w_ref[...], dn, preferred_element_type=jnp.float32)
    o_ref[...] = y + b_ref[...]


def _tc_final(pooled2, lin_W, lin_b):
    return pl.pallas_call(
        _tc_final_body,
        out_shape=jax.ShapeDtypeStruct((G, C), jnp.float32),
    )(pooled2, lin_W, lin_b.reshape(1, C))


def kernel(x, edge_index, batch, W1l, b1, W1r, W2l, b2, W2r, W3l, b3, W3r,
           W4l, b4, W4r, W5l, b5, W5r, W6l, b6, W6r, W7l, b7, W7r, lin_W, lin_b):
    src = edge_index[0]
    dst = edge_index[1]
    pad = E_PAD - E
    src_p = jnp.concatenate([src, jnp.zeros((pad,), jnp.int32)])
    dst_p = jnp.concatenate([dst, jnp.full((pad,), N, jnp.int32)])
    ids = (jnp.stack([src_p, dst_p], axis=0)
           .reshape(2, NCHT, CH).transpose(1, 0, 2))
    zeros = jnp.zeros((N_PAD, D), jnp.float32)
    ones = jnp.ones((CH, D), jnp.float32)

    cnt2 = _sc_cnt(ids, ones, zeros)
    inv = _tc_inv(cnt2)

    convs = [(W1l, b1, W1r), (W2l, b2, W2r), (W3l, b3, W3r), (W4l, b4, W4r),
             (W5l, b5, W5r), (W6l, b6, W6r), (W7l, b7, W7r)]
    h = x
    for i, (Wl, b, Wr) in enumerate(convs):
        r = _tc_right(h, Wr, b)
        agg2 = _sc_agg(h, ids, zeros)
        h = _tc_combine(agg2, inv, Wl, r, relu=(i < 6))

    pooled2 = _sc_pool(h, batch, zeros)
    return _tc_final(pooled2, lin_W, lin_b)
